# baseline (speedup 1.0000x reference)
import functools

import jax
import jax.numpy as jnp
from jax import lax
from jax.experimental import pallas as pl
from jax.experimental.pallas import tpu as pltpu
from jax.experimental.pallas import tpu_sc as plsc

_N = 256
_CIN = 5
_COUT = 10
_SLAB = (64, 256)

_BLOCKS_IN = _N * _CIN
_BLOCKS_OUT = _N * _COUT

_NC = 2
_NS = 16
_NW = _NC * _NS
_SLABS_PER_W = _BLOCKS_IN // _NW
_NBUF = 7
_AHEAD = 3

_mesh = plsc.VectorSubcoreMesh(core_axis_name="c", subcore_axis_name="s")


@functools.partial(
    pl.kernel,
    out_type=jax.ShapeDtypeStruct((_BLOCKS_OUT,) + _SLAB, jnp.float32),
    mesh=_mesh,
    scratch_types=[
        pltpu.VMEM((_NBUF,) + _SLAB, jnp.float32),
        pltpu.SemaphoreType.DMA((_NBUF,)),
        pltpu.SemaphoreType.DMA((_NBUF,)),
    ],
    compiler_params=pltpu.CompilerParams(use_tc_tiling_on_sc=True),
)
def _dup_slabs(in_hbm, out_hbm, buf, in_sem, out_sem):
    wid = lax.axis_index("s") * _NC + lax.axis_index("c")
    base = wid * _SLABS_PER_W

    def fetch(s, slot):
        pltpu.async_copy(in_hbm.at[base + s], buf.at[slot], in_sem.at[slot])

    def in_wait(slot):
        pltpu.make_async_copy(in_hbm.at[0], buf.at[slot],
                              in_sem.at[slot]).wait()

    def put(s, slot):
        o = 2 * (base + s)
        pltpu.async_copy(buf.at[slot], out_hbm.at[o], out_sem.at[slot])
        pltpu.async_copy(buf.at[slot], out_hbm.at[o + 1], out_sem.at[slot])

    def out_wait(slot):
        pltpu.make_async_copy(buf.at[slot], out_hbm.at[0],
                              out_sem.at[slot]).wait()
        pltpu.make_async_copy(buf.at[slot], out_hbm.at[0],
                              out_sem.at[slot]).wait()

    for b in range(_AHEAD):
        fetch(b, b)

    def step(s, _):
        slot = lax.rem(s, _NBUF)
        ahead = s + _AHEAD

        @pl.when(ahead < _SLABS_PER_W)
        def _():
            @pl.when(ahead >= _NBUF)
            def _():
                out_wait(lax.rem(ahead, _NBUF))
            fetch(ahead, lax.rem(ahead, _NBUF))

        in_wait(slot)
        put(s, slot)
        return ()

    lax.fori_loop(0, _SLABS_PER_W, step, ())
    for b in range(_NBUF):
        out_wait(b)


def kernel(body):
    bt = jnp.transpose(body, (0, 3, 2, 1)).reshape((_BLOCKS_IN,) + _SLAB)
    out_t = _dup_slabs(bt)
    out4 = out_t.reshape(_N, _COUT, _SLAB[0], _SLAB[1])
    return jnp.transpose(out4, (0, 3, 2, 1))


# speedup vs baseline: 1.0078x; 1.0078x over previous
import functools

import jax
import jax.numpy as jnp
from jax import lax
from jax.experimental import pallas as pl
from jax.experimental.pallas import tpu as pltpu
from jax.experimental.pallas import tpu_sc as plsc

_N = 256
_CIN = 5
_COUT = 10
_SLAB = (64, 256)

_BLOCKS_IN = _N * _CIN
_BLOCKS_OUT = _N * _COUT
_PAIRS = _BLOCKS_IN // 2

_NC = 2
_NS = 16
_NW = _NC * _NS
_PAIRS_PER_W = _PAIRS // _NW
_NBUF = 3
_AHEAD = 1

_mesh = plsc.VectorSubcoreMesh(core_axis_name="c", subcore_axis_name="s")


@functools.partial(
    pl.kernel,
    out_type=jax.ShapeDtypeStruct((_BLOCKS_OUT,) + _SLAB, jnp.float32),
    mesh=_mesh,
    scratch_types=[
        pltpu.VMEM((_NBUF, 2) + _SLAB, jnp.float32),
        pltpu.SemaphoreType.DMA((_NBUF,)),
        pltpu.SemaphoreType.DMA((_NBUF,)),
    ],
    compiler_params=pltpu.CompilerParams(use_tc_tiling_on_sc=True),
)
def _dup_slabs(in_hbm, out_hbm, buf, in_sem, out_sem):
    wid = lax.axis_index("s") * _NC + lax.axis_index("c")
    base = wid * _PAIRS_PER_W

    def fetch(p, slot):
        pltpu.async_copy(in_hbm.at[pl.ds(2 * (base + p), 2)], buf.at[slot],
                         in_sem.at[slot])

    def in_wait(slot):
        pltpu.make_async_copy(in_hbm.at[pl.ds(0, 2)], buf.at[slot],
                              in_sem.at[slot]).wait()

    def put(p, slot):
        o = 4 * (base + p)
        pltpu.async_copy(buf.at[slot, 0], out_hbm.at[o], out_sem.at[slot])
        pltpu.async_copy(buf.at[slot], out_hbm.at[pl.ds(o + 1, 2)],
                         out_sem.at[slot])
        pltpu.async_copy(buf.at[slot, 1], out_hbm.at[o + 3], out_sem.at[slot])

    def out_wait(slot):
        pltpu.make_async_copy(buf.at[slot], out_hbm.at[pl.ds(0, 2)],
                              out_sem.at[slot]).wait()
        pltpu.make_async_copy(buf.at[slot, 0], out_hbm.at[0],
                              out_sem.at[slot]).wait()
        pltpu.make_async_copy(buf.at[slot, 1], out_hbm.at[0],
                              out_sem.at[slot]).wait()

    for b in range(_AHEAD):
        fetch(b, b)

    def step(p, _):
        slot = lax.rem(p, _NBUF)
        ahead = p + _AHEAD

        @pl.when(ahead < _PAIRS_PER_W)
        def _():
            @pl.when(ahead >= _NBUF)
            def _():
                out_wait(lax.rem(ahead, _NBUF))
            fetch(ahead, lax.rem(ahead, _NBUF))

        in_wait(slot)
        put(p, slot)
        return ()

    lax.fori_loop(0, _PAIRS_PER_W, step, ())
    for b in range(_NBUF):
        out_wait(b)


def kernel(body):
    bt = jnp.transpose(body, (0, 3, 2, 1)).reshape((_BLOCKS_IN,) + _SLAB)
    out_t = _dup_slabs(bt)
    out4 = out_t.reshape(_N, _COUT, _SLAB[0], _SLAB[1])
    return jnp.transpose(out4, (0, 3, 2, 1))


# speedup vs baseline: 1.0124x; 1.0046x over previous
import functools

import jax
import jax.numpy as jnp
from jax import lax
from jax.experimental import pallas as pl
from jax.experimental.pallas import tpu as pltpu
from jax.experimental.pallas import tpu_sc as plsc

_N = 256
_CIN = 5
_COUT = 10
_SLAB = (64, 256)

_BLOCKS_IN = _N * _CIN
_BLOCKS_OUT = _N * _COUT
_PAIRS = _BLOCKS_IN // 2

_NC = 2
_NS = 16
_NW = _NC * _NS
_PAIRS_PER_W = _PAIRS // _NW
_NBUF = 3
_AHEAD = 1

_mesh = plsc.VectorSubcoreMesh(core_axis_name="c", subcore_axis_name="s")


@functools.partial(
    pl.kernel,
    out_type=jax.ShapeDtypeStruct((_BLOCKS_OUT,) + _SLAB, jnp.float32),
    mesh=_mesh,
    scratch_types=[
        pltpu.VMEM_SHARED((_NS, _NBUF, 2) + _SLAB, jnp.float32),
        pltpu.SemaphoreType.DMA((_NBUF,)),
        pltpu.SemaphoreType.DMA((_NBUF,)),
    ],
    compiler_params=pltpu.CompilerParams(use_tc_tiling_on_sc=True),
)
def _dup_slabs(in_hbm, out_hbm, shbuf, in_sem, out_sem):
    wid = lax.axis_index("s") * _NC + lax.axis_index("c")
    base = wid * _PAIRS_PER_W
    buf = shbuf.at[lax.axis_index("s")]

    def fetch(p, slot):
        pltpu.async_copy(in_hbm.at[pl.ds(2 * (base + p), 2)], buf.at[slot],
                         in_sem.at[slot])

    def in_wait(slot):
        pltpu.make_async_copy(in_hbm.at[pl.ds(0, 2)], buf.at[slot],
                              in_sem.at[slot]).wait()

    def put(p, slot):
        o = 4 * (base + p)
        pltpu.async_copy(buf.at[slot, 0], out_hbm.at[o], out_sem.at[slot])
        pltpu.async_copy(buf.at[slot], out_hbm.at[pl.ds(o + 1, 2)],
                         out_sem.at[slot])
        pltpu.async_copy(buf.at[slot, 1], out_hbm.at[o + 3], out_sem.at[slot])

    def out_wait(slot):
        pltpu.make_async_copy(buf.at[slot], out_hbm.at[pl.ds(0, 2)],
                              out_sem.at[slot]).wait()
        pltpu.make_async_copy(buf.at[slot, 0], out_hbm.at[0],
                              out_sem.at[slot]).wait()
        pltpu.make_async_copy(buf.at[slot, 1], out_hbm.at[0],
                              out_sem.at[slot]).wait()

    for b in range(_AHEAD):
        fetch(b, b)

    def step(p, _):
        slot = lax.rem(p, _NBUF)
        ahead = p + _AHEAD

        @pl.when(ahead < _PAIRS_PER_W)
        def _():
            @pl.when(ahead >= _NBUF)
            def _():
                out_wait(lax.rem(ahead, _NBUF))
            fetch(ahead, lax.rem(ahead, _NBUF))

        in_wait(slot)
        put(p, slot)
        return ()

    lax.fori_loop(0, _PAIRS_PER_W, step, ())
    for b in range(_NBUF):
        out_wait(b)


def kernel(body):
    bt = jnp.transpose(body, (0, 3, 2, 1)).reshape((_BLOCKS_IN,) + _SLAB)
    out_t = _dup_slabs(bt)
    out4 = out_t.reshape(_N, _COUT, _SLAB[0], _SLAB[1])
    return jnp.transpose(out4, (0, 3, 2, 1))


# speedup vs baseline: 1.0175x; 1.0050x over previous
import functools

import jax
import jax.numpy as jnp
from jax import lax
from jax.experimental import pallas as pl
from jax.experimental.pallas import tpu as pltpu
from jax.experimental.pallas import tpu_sc as plsc

_N = 256
_CIN = 5
_COUT = 10
_SLAB = (64, 256)

_BLOCKS_IN = _N * _CIN
_BLOCKS_OUT = _N * _COUT
_PAIRS = _BLOCKS_IN // 2

_NC = 2
_NS = 16
_NW = _NC * _NS
_PAIRS_PER_W = _PAIRS // _NW
_STEPS = _PAIRS_PER_W // 2
_NBUF = 2

_mesh = plsc.VectorSubcoreMesh(core_axis_name="c", subcore_axis_name="s")


@functools.partial(
    pl.kernel,
    out_type=jax.ShapeDtypeStruct((_BLOCKS_OUT,) + _SLAB, jnp.float32),
    mesh=_mesh,
    scratch_types=[
        pltpu.VMEM((_NBUF, 2) + _SLAB, jnp.float32),
        pltpu.VMEM_SHARED((_NS, _NBUF, 2) + _SLAB, jnp.float32),
        pltpu.SemaphoreType.DMA((_NBUF,)),
        pltpu.SemaphoreType.DMA((_NBUF,)),
        pltpu.SemaphoreType.DMA((_NBUF,)),
        pltpu.SemaphoreType.DMA((_NBUF,)),
    ],
    compiler_params=pltpu.CompilerParams(use_tc_tiling_on_sc=True),
)
def _dup_slabs(in_hbm, out_hbm, tbuf, shbuf, tin_sem, tout_sem,
               sin_sem, sout_sem):
    wid = lax.axis_index("s") * _NC + lax.axis_index("c")
    base = wid * _PAIRS_PER_W
    sbuf = shbuf.at[lax.axis_index("s")]

    def make_ring(buf, in_sem, out_sem, pair_of):
        def fetch(t, slot):
            pltpu.async_copy(in_hbm.at[pl.ds(2 * pair_of(t), 2)],
                             buf.at[slot], in_sem.at[slot])

        def in_wait(slot):
            pltpu.make_async_copy(in_hbm.at[pl.ds(0, 2)], buf.at[slot],
                                  in_sem.at[slot]).wait()

        def put(t, slot):
            o = 4 * pair_of(t)
            pltpu.async_copy(buf.at[slot, 0], out_hbm.at[o],
                             out_sem.at[slot])
            pltpu.async_copy(buf.at[slot], out_hbm.at[pl.ds(o + 1, 2)],
                             out_sem.at[slot])
            pltpu.async_copy(buf.at[slot, 1], out_hbm.at[o + 3],
                             out_sem.at[slot])

        def out_wait(slot):
            pltpu.make_async_copy(buf.at[slot], out_hbm.at[pl.ds(0, 2)],
                                  out_sem.at[slot]).wait()
            pltpu.make_async_copy(buf.at[slot, 0], out_hbm.at[0],
                                  out_sem.at[slot]).wait()
            pltpu.make_async_copy(buf.at[slot, 1], out_hbm.at[0],
                                  out_sem.at[slot]).wait()

        return fetch, in_wait, put, out_wait

    tfetch, twait, tput, tdrain = make_ring(
        tbuf, tin_sem, tout_sem, lambda t: base + 2 * t)
    sfetch, swait, sput, sdrain = make_ring(
        sbuf, sin_sem, sout_sem, lambda t: base + 2 * t + 1)

    tfetch(0, 0)
    sfetch(0, 0)

    def step(t, _):
        slot = lax.rem(t, _NBUF)
        ahead = t + 1

        @pl.when(ahead < _STEPS)
        def _():
            @pl.when(ahead >= _NBUF)
            def _():
                tdrain(lax.rem(ahead, _NBUF))
                sdrain(lax.rem(ahead, _NBUF))
            tfetch(ahead, lax.rem(ahead, _NBUF))
            sfetch(ahead, lax.rem(ahead, _NBUF))

        twait(slot)
        tput(t, slot)
        swait(slot)
        sput(t, slot)
        return ()

    lax.fori_loop(0, _STEPS, step, ())
    for b in range(_NBUF):
        tdrain(b)
        sdrain(b)


def kernel(body):
    bt = jnp.transpose(body, (0, 3, 2, 1)).reshape((_BLOCKS_IN,) + _SLAB)
    out_t = _dup_slabs(bt)
    out4 = out_t.reshape(_N, _COUT, _SLAB[0], _SLAB[1])
    return jnp.transpose(out4, (0, 3, 2, 1))
